# baseline (speedup 1.0000x reference)
import functools

import jax
import jax.numpy as jnp
from jax import lax
from jax.experimental import pallas as pl
from jax.experimental.pallas import tpu as pltpu
from jax.experimental.pallas import tpu_sc as plsc

F = 20
K = 32
B = 1024
L = 16
NC, NS = 2, 16
NW = NC * NS
BPW = B // NW
WS = 24


def _sc_gather(user, product, uf_t, pf):
  mesh = plsc.VectorSubcoreMesh(core_axis_name="c", subcore_axis_name="s")

  @functools.partial(
      pl.kernel,
      mesh=mesh,
      compiler_params=pltpu.CompilerParams(
          use_tc_tiling_on_sc=True, needs_layout_passes=False),
      out_type=[
          jax.ShapeDtypeStruct((B, K), jnp.float32),
          jax.ShapeDtypeStruct((B, K), jnp.float32),
      ],
      scratch_types=[
          pltpu.VMEM((BPW,), jnp.int32),
          pltpu.VMEM((BPW,), jnp.int32),
          pltpu.VMEM((BPW * WS, 128), jnp.float32),
          pltpu.VMEM((BPW, K), jnp.float32),
          pltpu.VMEM((BPW, K), jnp.float32),
          pltpu.SemaphoreType.DMA,
      ],
  )
  def gather_kernel(user_hbm, prod_hbm, uft_hbm, pf_hbm, u_out, p_out,
                    uidx_v, pidx_v, win_v, u_slab, p_slab, sem):
    c = lax.axis_index("c")
    s = lax.axis_index("s")
    base = (c * NS + s) * BPW
    pltpu.sync_copy(user_hbm.at[pl.ds(base, BPW)], uidx_v)
    pltpu.sync_copy(prod_hbm.at[pl.ds(base, BPW)], pidx_v)

    iota = lax.iota(jnp.int32, L)
    zeros = jnp.zeros((L,), jnp.float32)
    iota_ws = iota * WS

    def gather_one(idx_v, table_hbm, slab_v):
      @pl.loop(0, BPW // L)
      def _(i):
        part = idx_v[pl.ds(i * L, L)]
        hi = lax.bitwise_and(part, -128)
        for j in range(L):
          col = pl.multiple_of(hi[j], 128)
          pltpu.async_copy(
              table_hbm.at[:, pl.ds(col, 128)],
              win_v.at[pl.ds((i * L + j) * WS, F)], sem)

      @pl.loop(0, BPW)
      def _(b):
        pltpu.make_async_copy(
            table_hbm.at[:, pl.ds(0, 128)], win_v.at[pl.ds(0, F)], sem
        ).wait()

      @pl.loop(0, BPW // L)
      def _(i):
        lo = lax.bitwise_and(idx_v[pl.ds(i * L, L)], 127)
        bvec = i * L + iota
        for k in range(K):
          if k < F:
            vals = plsc.load_gather(win_v, [iota_ws + (i * L * WS + k), lo])
          else:
            vals = zeros
          plsc.store_scatter(slab_v, [bvec, jnp.full((L,), k, jnp.int32)],
                             vals)

    gather_one(uidx_v, uft_hbm, u_slab)
    pltpu.sync_copy(u_slab, u_out.at[pl.ds(base, BPW)])
    gather_one(pidx_v, pf_hbm, p_slab)
    pltpu.sync_copy(p_slab, p_out.at[pl.ds(base, BPW)])

  return gather_kernel(user, product, uf_t, pf)


def _mm_body(u_ref, p_ref, o_ref):
  o_ref[...] = lax.dot_general(
      u_ref[...], p_ref[...], (((1,), (1,)), ((), ())),
      preferred_element_type=jnp.float32)


def _tc_matmul(u, p):
  bm = 128
  return pl.pallas_call(
      _mm_body,
      grid=(B // bm,),
      in_specs=[
          pl.BlockSpec((bm, K), lambda i: (i, 0)),
          pl.BlockSpec((B, K), lambda i: (0, 0)),
      ],
      out_specs=pl.BlockSpec((bm, B), lambda i: (i, 0)),
      out_shape=jax.ShapeDtypeStruct((B, B), jnp.float32),
  )(u, p)


def kernel(user, product, user_factors, product_factors):
  u, p = _sc_gather(user, product, user_factors.T, product_factors)
  return _tc_matmul(u, p)


# speedup vs baseline: 1.0797x; 1.0797x over previous
import functools

import jax
import jax.numpy as jnp
from jax import lax
from jax.experimental import pallas as pl
from jax.experimental.pallas import tpu as pltpu
from jax.experimental.pallas import tpu_sc as plsc

F = 20
K = 32
B = 1024
L = 16
NC, NS = 2, 16
NW = NC * NS
BPW = B // NW
WS = 24


def _sc_gather(user, product, uf_t, pf):
  mesh = plsc.VectorSubcoreMesh(core_axis_name="c", subcore_axis_name="s")

  @functools.partial(
      pl.kernel,
      mesh=mesh,
      compiler_params=pltpu.CompilerParams(
          use_tc_tiling_on_sc=True, needs_layout_passes=False),
      out_type=[
          jax.ShapeDtypeStruct((B, K), jnp.float32),
          jax.ShapeDtypeStruct((B, K), jnp.float32),
      ],
      scratch_types=[
          pltpu.VMEM((BPW,), jnp.int32),
          pltpu.VMEM((BPW,), jnp.int32),
          pltpu.VMEM((BPW * WS, 128), jnp.float32),
          pltpu.VMEM((BPW, K), jnp.float32),
          pltpu.VMEM((BPW, K), jnp.float32),
          pltpu.SemaphoreType.DMA,
      ],
  )
  def gather_kernel(user_hbm, prod_hbm, uft_hbm, pf_hbm, u_out, p_out,
                    uidx_v, pidx_v, win_v, u_slab, p_slab, sem):
    c = lax.axis_index("c")
    s = lax.axis_index("s")
    base = (c * NS + s) * BPW
    pltpu.sync_copy(user_hbm.at[pl.ds(base, BPW)], uidx_v)
    pltpu.sync_copy(prod_hbm.at[pl.ds(base, BPW)], pidx_v)

    iota = lax.iota(jnp.int32, L)
    zeros = jnp.zeros((L,), jnp.float32)
    iota_ws = iota * WS

    def gather_one(idx_v, table_hbm, slab_v):
      @pl.loop(0, BPW // L)
      def _(i):
        part = idx_v[pl.ds(i * L, L)]
        hi = lax.bitwise_and(part, -128)
        for j in range(L):
          col = pl.multiple_of(hi[j], 128)
          pltpu.async_copy(
              table_hbm.at[:, pl.ds(col, 128)],
              win_v.at[pl.ds((i * L + j) * WS, F)], sem)

      @pl.loop(0, BPW)
      def _(b):
        pltpu.make_async_copy(
            table_hbm.at[:, pl.ds(0, 128)], win_v.at[pl.ds(0, F)], sem
        ).wait()

      @pl.loop(0, BPW // L)
      def _(i):
        lo = lax.bitwise_and(idx_v[pl.ds(i * L, L)], 127)
        bvec = i * L + iota
        for k in range(K):
          if k < F:
            vals = plsc.load_gather(win_v, [iota_ws + (i * L * WS + k), lo])
          else:
            vals = zeros
          plsc.store_scatter(slab_v, [bvec, jnp.full((L,), k, jnp.int32)],
                             vals)

    gather_one(uidx_v, uft_hbm, u_slab)
    pltpu.sync_copy(u_slab, u_out.at[pl.ds(base, BPW)])
    gather_one(pidx_v, pf_hbm, p_slab)
    pltpu.sync_copy(p_slab, p_out.at[pl.ds(base, BPW)])

  return gather_kernel(user, product, uf_t, pf)


def _mm_body(u_ref, p_ref, o_ref):
  o_ref[...] = lax.dot_general(
      u_ref[...], p_ref[...], (((1,), (1,)), ((), ())),
      preferred_element_type=jnp.float32)


def _tc_matmul(u, p):
  bm = 512
  return pl.pallas_call(
      _mm_body,
      grid=(B // bm,),
      in_specs=[
          pl.BlockSpec((bm, K), lambda i: (i, 0)),
          pl.BlockSpec((B, K), lambda i: (0, 0)),
      ],
      out_specs=pl.BlockSpec((bm, B), lambda i: (i, 0)),
      out_shape=jax.ShapeDtypeStruct((B, B), jnp.float32),
  )(u, p)


def kernel(user, product, user_factors, product_factors):
  u, p = _sc_gather(user, product, user_factors.T, product_factors)
  return _tc_matmul(u, p)
